# Initial kernel scaffold; baseline (speedup 1.0000x reference)
import jax
import jax.numpy as jnp
from jax.experimental import pallas as pl


def kernel(img, shapes, queries, reference_points, Wi, bi, Wq, bq, Wo, bo):
    raise NotImplementedError("write your pallas kernel here")


# speedup vs baseline: 3266.7540x; 3266.7540x over previous
import functools

import jax
import jax.numpy as jnp
import numpy as np
from jax import lax
from jax.experimental import pallas as pl
from jax.experimental.pallas import tpu as pltpu
from jax.experimental.pallas import tpu_sc as plsc

EMB = 512
HID = 512
NHEADS = 16
NLEVELS = 4
NPOINTS = 8
B = 2
NQ = 1024
LEVEL_SHAPES = [[64, 64], [32, 32], [16, 16], [8, 8]]
I_TOTAL = sum(h * w for h, w in LEVEL_SHAPES)
C = HID // NHEADS
LANES = NHEADS * NLEVELS * NPOINTS
NPTS = B * NQ
NROWS = 4 * LANES
NWORK = 32
PTS_PER_W = NPTS // NWORK

_shapes_np = np.array(LEVEL_SHAPES, np.int32)
_sizes = _shapes_np[:, 0] * _shapes_np[:, 1]
_lev_start = np.concatenate([[0], np.cumsum(_sizes)[:-1]]).astype(np.int32)
_lane_l = (np.arange(LANES) // NPOINTS) % NLEVELS
_lane_h = np.arange(LANES) // (NLEVELS * NPOINTS)
_WM1 = (_shapes_np[_lane_l, 1] - 1).astype(np.float32)
_HM1 = (_shapes_np[_lane_l, 0] - 1).astype(np.float32)
_WVEC = _shapes_np[_lane_l, 1].astype(np.int32)
_COMB = (_lane_h * I_TOTAL + _lev_start[_lane_l]).astype(np.int32)


def _imgp_body(img_ref, wi_ref, bi_ref, out_ref):
    x = img_ref[0]
    y = jnp.dot(x, wi_ref[...], preferred_element_type=jnp.float32)
    y = y + bi_ref[0][None, :]
    for h in range(NHEADS):
        out_ref[0, h, :, :] = y[:, h * C:(h + 1) * C]


def _imgp_table(img, Wi, bi):
    IB = 544
    grid = (B, I_TOTAL // IB)
    return pl.pallas_call(
        _imgp_body,
        grid=grid,
        in_specs=[
            pl.BlockSpec((1, IB, EMB), lambda b, i: (b, i, 0)),
            pl.BlockSpec((EMB, HID), lambda b, i: (0, 0)),
            pl.BlockSpec((1, HID), lambda b, i: (0, 0)),
        ],
        out_specs=pl.BlockSpec((1, NHEADS, IB, C), lambda b, i: (b, 0, i, 0)),
        out_shape=jax.ShapeDtypeStruct((B, NHEADS, I_TOTAL, C), jnp.float32),
    )(img, Wi, bi.reshape(1, HID))


def _points_body(q_ref, rpx_ref, rpy_ref, wx_ref, wy_ref, wl_ref, bx_ref,
                 by_ref, bl_ref, wm1_ref, hm1_ref, wvec_ref, comb_ref,
                 idx_ref, wts_ref):
    b = pl.program_id(0)
    q = q_ref[0]
    rpx_row = rpx_ref[0, 0]
    rpy_row = rpy_ref[0, 0]
    ox = jnp.dot(q, wx_ref[...], preferred_element_type=jnp.float32) + bx_ref[0][None, :]
    oy = jnp.dot(q, wy_ref[...], preferred_element_type=jnp.float32) + by_ref[0][None, :]
    lg = jnp.dot(q, wl_ref[...], preferred_element_type=jnp.float32) + bl_ref[0][None, :]
    QB = ox.shape[0]
    lg3 = lg.reshape(QB, NHEADS, NLEVELS * NPOINTS)
    m = jnp.max(lg3, axis=2, keepdims=True)
    e = jnp.exp(lg3 - m)
    aw = (e / jnp.sum(e, axis=2, keepdims=True)).reshape(QB, LANES)

    wm1 = wm1_ref[0][None, :]
    hm1 = hm1_ref[0][None, :]
    spx = rpx_row[:, None] + ox
    spy = rpy_row[:, None] + oy
    x = jnp.clip(spx * wm1, 0.0, wm1)
    y = jnp.clip(spy * hm1, 0.0, hm1)
    x0f = jnp.floor(x)
    y0f = jnp.floor(y)
    fx = x - x0f
    fy = y - y0f
    x0 = x0f.astype(jnp.int32)
    y0 = y0f.astype(jnp.int32)
    x1 = jnp.minimum(x0 + 1, wm1.astype(jnp.int32))
    y1 = jnp.minimum(y0 + 1, hm1.astype(jnp.int32))
    wvec = wvec_ref[0][None, :]
    base = b * (NHEADS * I_TOTAL) + comb_ref[0][None, :]
    i00 = base + y0 * wvec + x0
    i01 = base + y0 * wvec + x1
    i10 = base + y1 * wvec + x0
    i11 = base + y1 * wvec + x1
    gx = 1.0 - fx
    gy = 1.0 - fy
    w00 = aw * gy * gx
    w01 = aw * gy * fx
    w10 = aw * fy * gx
    w11 = aw * fy * fx
    for t, (iv, wv) in enumerate(((i00, w00), (i01, w01), (i10, w10), (i11, w11))):
        for k in range(4):
            idx_ref[0, :, t * 4 + k, :] = iv[:, k * 128:(k + 1) * 128]
        wts_ref[0, :, pl.ds(t * LANES, LANES)] = wv


def _points(queries, reference_points, Wq, bq):
    QB = 128
    Wq3 = Wq.reshape(EMB, LANES, 3)
    Wx = Wq3[..., 0]
    Wy = Wq3[..., 1]
    Wl = Wq3[..., 2]
    bq3 = bq.reshape(LANES, 3)
    bx = bq3[:, 0].reshape(1, LANES)
    by = bq3[:, 1].reshape(1, LANES)
    bl = bq3[:, 2].reshape(1, LANES)
    rpx = reference_points[..., 0].reshape(B * (NQ // QB), 1, QB)
    rpy = reference_points[..., 1].reshape(B * (NQ // QB), 1, QB)
    grid = (B, NQ // QB)
    full = lambda b, i: (0, 0)
    idx, wts = pl.pallas_call(
        _points_body,
        grid=grid,
        in_specs=[
            pl.BlockSpec((1, QB, EMB), lambda b, i: (b, i, 0)),
            pl.BlockSpec((1, 1, QB), lambda b, i: (b * (NQ // QB) + i, 0, 0)),
            pl.BlockSpec((1, 1, QB), lambda b, i: (b * (NQ // QB) + i, 0, 0)),
            pl.BlockSpec((EMB, LANES), full),
            pl.BlockSpec((EMB, LANES), full),
            pl.BlockSpec((EMB, LANES), full),
            pl.BlockSpec((1, LANES), full),
            pl.BlockSpec((1, LANES), full),
            pl.BlockSpec((1, LANES), full),
            pl.BlockSpec((1, LANES), full),
            pl.BlockSpec((1, LANES), full),
            pl.BlockSpec((1, LANES), full),
            pl.BlockSpec((1, LANES), full),
        ],
        out_specs=[
            pl.BlockSpec((1, QB, 16, 128), lambda b, i: (b, i, 0, 0)),
            pl.BlockSpec((1, QB, NROWS), lambda b, i: (b, i, 0)),
        ],
        out_shape=[
            jax.ShapeDtypeStruct((B, NQ, 16, 128), jnp.int32),
            jax.ShapeDtypeStruct((B, NQ, NROWS), jnp.float32),
        ],
    )(queries, rpx, rpy, Wx, Wy, Wl, bx, by, bl,
      jnp.asarray(_WM1).reshape(1, LANES), jnp.asarray(_HM1).reshape(1, LANES),
      jnp.asarray(_WVEC).reshape(1, LANES), jnp.asarray(_COMB).reshape(1, LANES))
    return idx.reshape(NPTS, 16, 128), wts.reshape(NPTS, NROWS)


def _sc_body(table_hbm, idx_hbm, wts_hbm, out_hbm, idxv, wtsv, rowsv, outv, sem):
    wid = lax.axis_index("s") * 2 + lax.axis_index("c")
    base = wid * PTS_PER_W

    def point_body(i, _):
        pt = base + i
        pltpu.sync_copy(idx_hbm.at[pt], idxv)
        copies = [
            pltpu.make_async_copy(
                table_hbm.at[idxv.at[j]],
                rowsv.at[pl.ds(j * 128, 128), :],
                sem,
            )
            for j in range(16)
        ]
        for cp in copies:
            cp.start()
        pltpu.sync_copy(wts_hbm.at[pt], wtsv)
        for cp in copies:
            cp.wait()

        def h_body(h, _):
            hb = h * (NLEVELS * NPOINTS)
            acc = [jnp.zeros((16,), jnp.float32) for _ in range(4)]
            for t in range(4):
                for g in range(2):
                    jb = t * LANES + hb + g * 16
                    wv = wtsv[pl.ds(jb, 16)]
                    for e in range(16):
                        w = wv[e]
                        j = jb + e
                        r0 = rowsv[j, pl.ds(0, 16)]
                        r1 = rowsv[j, pl.ds(16, 16)]
                        k = (e & 1) * 2
                        acc[k] = acc[k] + w * r0
                        acc[k + 1] = acc[k + 1] + w * r1
            outv[pl.ds(h * C, 16)] = acc[0] + acc[2]
            outv[pl.ds(h * C + 16, 16)] = acc[1] + acc[3]
            return 0

        lax.fori_loop(0, NHEADS, h_body, 0)
        pltpu.sync_copy(outv, out_hbm.at[pt])
        return 0

    lax.fori_loop(0, PTS_PER_W, point_body, 0)


def _sc_gather(table, idx, wts):
    mesh = plsc.VectorSubcoreMesh(core_axis_name="c", subcore_axis_name="s")
    f = functools.partial(
        pl.kernel,
        mesh=mesh,
        compiler_params=pltpu.CompilerParams(use_tc_tiling_on_sc=False),
        out_type=jax.ShapeDtypeStruct((NPTS, HID), jnp.float32),
        scratch_types=[
            pltpu.VMEM((16, 128), jnp.int32),
            pltpu.VMEM((NROWS,), jnp.float32),
            pltpu.VMEM((NROWS, C), jnp.float32),
            pltpu.VMEM((HID,), jnp.float32),
            pltpu.SemaphoreType.DMA,
        ],
    )(_sc_body)
    return f(table.reshape(B * NHEADS * I_TOTAL, C), idx, wts)


def _proj_body(x_ref, wo_ref, bo_ref, out_ref):
    out_ref[...] = (jnp.dot(x_ref[...], wo_ref[...],
                            preferred_element_type=jnp.float32)
                    + bo_ref[0][None, :])


def _out_proj(acc, Wo, bo):
    MB = 256
    return pl.pallas_call(
        _proj_body,
        grid=(NPTS // MB,),
        in_specs=[
            pl.BlockSpec((MB, HID), lambda i: (i, 0)),
            pl.BlockSpec((HID, EMB), lambda i: (0, 0)),
            pl.BlockSpec((1, EMB), lambda i: (0, 0)),
        ],
        out_specs=pl.BlockSpec((MB, EMB), lambda i: (i, 0)),
        out_shape=jax.ShapeDtypeStruct((NPTS, EMB), jnp.float32),
    )(acc, Wo, bo.reshape(1, EMB))


def kernel(img, shapes, queries, reference_points, Wi, bi, Wq, bq, Wo, bo):
    table = _imgp_table(img, Wi, bi)
    idx, wts = _points(queries, reference_points, Wq, bq)
    acc = _sc_gather(table, idx, wts)
    out = _out_proj(acc, Wo, bo)
    return out.reshape(B, NQ, EMB)


# speedup vs baseline: 3297.8451x; 1.0095x over previous
import functools

import jax
import jax.numpy as jnp
import numpy as np
from jax import lax
from jax.experimental import pallas as pl
from jax.experimental.pallas import tpu as pltpu
from jax.experimental.pallas import tpu_sc as plsc

EMB = 512
HID = 512
NHEADS = 16
NLEVELS = 4
NPOINTS = 8
B = 2
NQ = 1024
LEVEL_SHAPES = [[64, 64], [32, 32], [16, 16], [8, 8]]
I_TOTAL = sum(h * w for h, w in LEVEL_SHAPES)
C = HID // NHEADS
LANES = NHEADS * NLEVELS * NPOINTS
NPTS = B * NQ
NROWS = 4 * LANES
NWORK = 32
PTS_PER_W = NPTS // NWORK

_shapes_np = np.array(LEVEL_SHAPES, np.int32)
_sizes = _shapes_np[:, 0] * _shapes_np[:, 1]
_lev_start = np.concatenate([[0], np.cumsum(_sizes)[:-1]]).astype(np.int32)
_lane_l = (np.arange(LANES) // NPOINTS) % NLEVELS
_lane_h = np.arange(LANES) // (NLEVELS * NPOINTS)
_WM1 = (_shapes_np[_lane_l, 1] - 1).astype(np.float32)
_HM1 = (_shapes_np[_lane_l, 0] - 1).astype(np.float32)
_WVEC = _shapes_np[_lane_l, 1].astype(np.int32)
_COMB = (_lane_h * I_TOTAL + _lev_start[_lane_l]).astype(np.int32)


def _imgp_body(img_ref, wi_ref, bi_ref, out_ref):
    x = img_ref[0]
    y = jnp.dot(x, wi_ref[...], preferred_element_type=jnp.float32)
    y = y + bi_ref[0][None, :]
    yb = y.astype(jnp.bfloat16)
    for h in range(NHEADS):
        out_ref[0, h, :, :] = yb[:, h * C:(h + 1) * C]


def _imgp_table(img, Wi, bi):
    IB = 544
    grid = (B, I_TOTAL // IB)
    return pl.pallas_call(
        _imgp_body,
        grid=grid,
        in_specs=[
            pl.BlockSpec((1, IB, EMB), lambda b, i: (b, i, 0)),
            pl.BlockSpec((EMB, HID), lambda b, i: (0, 0)),
            pl.BlockSpec((1, HID), lambda b, i: (0, 0)),
        ],
        out_specs=pl.BlockSpec((1, NHEADS, IB, C), lambda b, i: (b, 0, i, 0)),
        out_shape=jax.ShapeDtypeStruct((B, NHEADS, I_TOTAL, C), jnp.bfloat16),
    )(img, Wi, bi.reshape(1, HID))


def _points_body(q_ref, rpx_ref, rpy_ref, wx_ref, wy_ref, wl_ref, bx_ref,
                 by_ref, bl_ref, wm1_ref, hm1_ref, wvec_ref, comb_ref,
                 idx_ref, wts_ref):
    b = pl.program_id(0)
    q = q_ref[0]
    rpx_row = rpx_ref[0, 0]
    rpy_row = rpy_ref[0, 0]
    ox = jnp.dot(q, wx_ref[...], preferred_element_type=jnp.float32) + bx_ref[0][None, :]
    oy = jnp.dot(q, wy_ref[...], preferred_element_type=jnp.float32) + by_ref[0][None, :]
    lg = jnp.dot(q, wl_ref[...], preferred_element_type=jnp.float32) + bl_ref[0][None, :]
    QB = ox.shape[0]
    lg3 = lg.reshape(QB, NHEADS, NLEVELS * NPOINTS)
    m = jnp.max(lg3, axis=2, keepdims=True)
    e = jnp.exp(lg3 - m)
    aw = (e / jnp.sum(e, axis=2, keepdims=True)).reshape(QB, LANES)

    wm1 = wm1_ref[0][None, :]
    hm1 = hm1_ref[0][None, :]
    spx = rpx_row[:, None] + ox
    spy = rpy_row[:, None] + oy
    x = jnp.clip(spx * wm1, 0.0, wm1)
    y = jnp.clip(spy * hm1, 0.0, hm1)
    x0f = jnp.floor(x)
    y0f = jnp.floor(y)
    fx = x - x0f
    fy = y - y0f
    x0 = x0f.astype(jnp.int32)
    y0 = y0f.astype(jnp.int32)
    x1 = jnp.minimum(x0 + 1, wm1.astype(jnp.int32))
    y1 = jnp.minimum(y0 + 1, hm1.astype(jnp.int32))
    wvec = wvec_ref[0][None, :]
    base = b * (NHEADS * I_TOTAL) + comb_ref[0][None, :]
    i00 = base + y0 * wvec + x0
    i01 = base + y0 * wvec + x1
    i10 = base + y1 * wvec + x0
    i11 = base + y1 * wvec + x1
    gx = 1.0 - fx
    gy = 1.0 - fy
    w00 = aw * gy * gx
    w01 = aw * gy * fx
    w10 = aw * fy * gx
    w11 = aw * fy * fx
    for t, (iv, wv) in enumerate(((i00, w00), (i01, w01), (i10, w10), (i11, w11))):
        for k in range(4):
            idx_ref[0, :, t * 4 + k, :] = iv[:, k * 128:(k + 1) * 128]
        wts_ref[0, :, pl.ds(t * LANES, LANES)] = wv


def _points(queries, reference_points, Wq, bq):
    QB = 128
    Wq3 = Wq.reshape(EMB, LANES, 3)
    Wx = Wq3[..., 0]
    Wy = Wq3[..., 1]
    Wl = Wq3[..., 2]
    bq3 = bq.reshape(LANES, 3)
    bx = bq3[:, 0].reshape(1, LANES)
    by = bq3[:, 1].reshape(1, LANES)
    bl = bq3[:, 2].reshape(1, LANES)
    rpx = reference_points[..., 0].reshape(B * (NQ // QB), 1, QB)
    rpy = reference_points[..., 1].reshape(B * (NQ // QB), 1, QB)
    grid = (B, NQ // QB)
    full = lambda b, i: (0, 0)
    idx, wts = pl.pallas_call(
        _points_body,
        grid=grid,
        in_specs=[
            pl.BlockSpec((1, QB, EMB), lambda b, i: (b, i, 0)),
            pl.BlockSpec((1, 1, QB), lambda b, i: (b * (NQ // QB) + i, 0, 0)),
            pl.BlockSpec((1, 1, QB), lambda b, i: (b * (NQ // QB) + i, 0, 0)),
            pl.BlockSpec((EMB, LANES), full),
            pl.BlockSpec((EMB, LANES), full),
            pl.BlockSpec((EMB, LANES), full),
            pl.BlockSpec((1, LANES), full),
            pl.BlockSpec((1, LANES), full),
            pl.BlockSpec((1, LANES), full),
            pl.BlockSpec((1, LANES), full),
            pl.BlockSpec((1, LANES), full),
            pl.BlockSpec((1, LANES), full),
            pl.BlockSpec((1, LANES), full),
        ],
        out_specs=[
            pl.BlockSpec((1, QB, 16, 128), lambda b, i: (b, i, 0, 0)),
            pl.BlockSpec((1, QB, NROWS), lambda b, i: (b, i, 0)),
        ],
        out_shape=[
            jax.ShapeDtypeStruct((B, NQ, 16, 128), jnp.int32),
            jax.ShapeDtypeStruct((B, NQ, NROWS), jnp.float32),
        ],
    )(queries, rpx, rpy, Wx, Wy, Wl, bx, by, bl,
      jnp.asarray(_WM1).reshape(1, LANES), jnp.asarray(_HM1).reshape(1, LANES),
      jnp.asarray(_WVEC).reshape(1, LANES), jnp.asarray(_COMB).reshape(1, LANES))
    return idx.reshape(NPTS, 16, 128), wts.reshape(NPTS, NROWS)


def _sc_body(table_hbm, idx_hbm, wts_hbm, out_hbm, idxv, wtsv, rowsv, outv, sem):
    wid = lax.axis_index("s") * 2 + lax.axis_index("c")
    base = wid * PTS_PER_W

    def point_body(i, _):
        pt = base + i
        pltpu.sync_copy(idx_hbm.at[pt], idxv)
        copies = [
            pltpu.make_async_copy(
                table_hbm.at[idxv.at[j]],
                rowsv.at[pl.ds(j * 128, 128), :],
                sem,
            )
            for j in range(16)
        ]
        for cp in copies:
            cp.start()
        pltpu.sync_copy(wts_hbm.at[pt], wtsv)
        for cp in copies:
            cp.wait()

        def h_body(h, _):
            hb = h * (NLEVELS * NPOINTS)
            acc = [jnp.zeros((16,), jnp.float32) for _ in range(4)]
            for t in range(4):
                for g in range(2):
                    jb = t * LANES + hb + g * 16
                    wv = wtsv[pl.ds(jb, 16)]
                    for e in range(16):
                        w = wv[e]
                        j = jb + e
                        rev, rod = plsc.unpack(
                            rowsv[j, :], format=plsc.PackFormat.INTERLEAVED)
                        k = (e & 1) * 2
                        acc[k] = acc[k] + w * rev
                        acc[k + 1] = acc[k + 1] + w * rod
            outv[pl.ds(h * C, 16)] = acc[0] + acc[2]
            outv[pl.ds(h * C + 16, 16)] = acc[1] + acc[3]
            return 0

        lax.fori_loop(0, NHEADS, h_body, 0)
        pltpu.sync_copy(outv, out_hbm.at[pt])
        return 0

    lax.fori_loop(0, PTS_PER_W, point_body, 0)


def _sc_gather(table, idx, wts):
    mesh = plsc.VectorSubcoreMesh(core_axis_name="c", subcore_axis_name="s")
    f = functools.partial(
        pl.kernel,
        mesh=mesh,
        compiler_params=pltpu.CompilerParams(use_tc_tiling_on_sc=False,
                                             needs_layout_passes=False),
        out_type=jax.ShapeDtypeStruct((NPTS, HID), jnp.float32),
        scratch_types=[
            pltpu.VMEM((16, 128), jnp.int32),
            pltpu.VMEM((NROWS,), jnp.float32),
            pltpu.VMEM((NROWS, C), jnp.bfloat16),
            pltpu.VMEM((HID,), jnp.float32),
            pltpu.SemaphoreType.DMA,
        ],
    )(_sc_body)
    return f(table.reshape(B * NHEADS * I_TOTAL, C), idx, wts)


def _proj_body(x_ref, wo_ref, bo_ref, out_ref):
    out_ref[...] = (jnp.dot(x_ref[...], wo_ref[...],
                            preferred_element_type=jnp.float32)
                    + bo_ref[0][None, :])


_kk = np.tile(np.arange(C), NHEADS)
_hh = np.repeat(np.arange(NHEADS), C) * C
_PERM = (_hh + np.where(_kk < 16, 2 * _kk, 2 * (_kk - 16) + 1)).astype(np.int32)


def _out_proj(acc, Wo, bo):
    MB = 256
    return pl.pallas_call(
        _proj_body,
        grid=(NPTS // MB,),
        in_specs=[
            pl.BlockSpec((MB, HID), lambda i: (i, 0)),
            pl.BlockSpec((HID, EMB), lambda i: (0, 0)),
            pl.BlockSpec((1, EMB), lambda i: (0, 0)),
        ],
        out_specs=pl.BlockSpec((MB, EMB), lambda i: (i, 0)),
        out_shape=jax.ShapeDtypeStruct((NPTS, EMB), jnp.float32),
    )(acc, Wo[jnp.asarray(_PERM)], bo.reshape(1, EMB))


def kernel(img, shapes, queries, reference_points, Wi, bi, Wq, bq, Wo, bo):
    table = _imgp_table(img, Wi, bi)
    idx, wts = _points(queries, reference_points, Wq, bq)
    acc = _sc_gather(table, idx, wts)
    out = _out_proj(acc, Wo, bo)
    return out.reshape(B, NQ, EMB)


# speedup vs baseline: 4748.5614x; 1.4399x over previous
import functools

import jax
import jax.numpy as jnp
import numpy as np
from jax import lax
from jax.experimental import pallas as pl
from jax.experimental.pallas import tpu as pltpu
from jax.experimental.pallas import tpu_sc as plsc

EMB = 512
HID = 512
NHEADS = 16
NLEVELS = 4
NPOINTS = 8
B = 2
NQ = 1024
LEVEL_SHAPES = [[64, 64], [32, 32], [16, 16], [8, 8]]
I_TOTAL = sum(h * w for h, w in LEVEL_SHAPES)
C = HID // NHEADS
LANES = NHEADS * NLEVELS * NPOINTS
NPTS = B * NQ
NROWS = 4 * LANES
NWORK = 32
PTS_PER_W = NPTS // NWORK

_shapes_np = np.array(LEVEL_SHAPES, np.int32)
_sizes = _shapes_np[:, 0] * _shapes_np[:, 1]
_lev_start = np.concatenate([[0], np.cumsum(_sizes)[:-1]]).astype(np.int32)
_lane_l = (np.arange(LANES) // NPOINTS) % NLEVELS
_lane_h = np.arange(LANES) // (NLEVELS * NPOINTS)
_WM1 = (_shapes_np[_lane_l, 1] - 1).astype(np.float32)
_HM1 = (_shapes_np[_lane_l, 0] - 1).astype(np.float32)
_WVEC = _shapes_np[_lane_l, 1].astype(np.int32)
_COMB = (_lane_h * I_TOTAL + _lev_start[_lane_l]).astype(np.int32)


def _imgp_body(img_ref, wi_ref, bi_ref, out_ref):
    x = img_ref[0]
    y = jnp.dot(x, wi_ref[...], preferred_element_type=jnp.float32)
    y = y + bi_ref[0][None, :]
    yb = y.astype(jnp.bfloat16)
    for h in range(NHEADS):
        out_ref[0, h, :, :] = yb[:, h * C:(h + 1) * C]


def _imgp_table(img, Wi, bi):
    IB = 544
    grid = (B, I_TOTAL // IB)
    return pl.pallas_call(
        _imgp_body,
        grid=grid,
        in_specs=[
            pl.BlockSpec((1, IB, EMB), lambda b, i: (b, i, 0)),
            pl.BlockSpec((EMB, HID), lambda b, i: (0, 0)),
            pl.BlockSpec((1, HID), lambda b, i: (0, 0)),
        ],
        out_specs=pl.BlockSpec((1, NHEADS, IB, C), lambda b, i: (b, 0, i, 0)),
        out_shape=jax.ShapeDtypeStruct((B, NHEADS, I_TOTAL, C), jnp.bfloat16),
    )(img, Wi, bi.reshape(1, HID))


def _points_body(q_ref, rpx_ref, rpy_ref, wx_ref, wy_ref, wl_ref, bx_ref,
                 by_ref, bl_ref, wm1_ref, hm1_ref, wvec_ref, comb_ref,
                 idx_ref, wts_ref):
    b = pl.program_id(0)
    q = q_ref[0]
    rpx_row = rpx_ref[0, 0]
    rpy_row = rpy_ref[0, 0]
    ox = jnp.dot(q, wx_ref[...], preferred_element_type=jnp.float32) + bx_ref[0][None, :]
    oy = jnp.dot(q, wy_ref[...], preferred_element_type=jnp.float32) + by_ref[0][None, :]
    lg = jnp.dot(q, wl_ref[...], preferred_element_type=jnp.float32) + bl_ref[0][None, :]
    QB = ox.shape[0]
    lg3 = lg.reshape(QB, NHEADS, NLEVELS * NPOINTS)
    m = jnp.max(lg3, axis=2, keepdims=True)
    e = jnp.exp(lg3 - m)
    aw = (e / jnp.sum(e, axis=2, keepdims=True)).reshape(QB, LANES)

    wm1 = wm1_ref[0][None, :]
    hm1 = hm1_ref[0][None, :]
    spx = rpx_row[:, None] + ox
    spy = rpy_row[:, None] + oy
    x = jnp.clip(spx * wm1, 0.0, wm1)
    y = jnp.clip(spy * hm1, 0.0, hm1)
    x0f = jnp.floor(x)
    y0f = jnp.floor(y)
    fx = x - x0f
    fy = y - y0f
    x0 = x0f.astype(jnp.int32)
    y0 = y0f.astype(jnp.int32)
    x1 = jnp.minimum(x0 + 1, wm1.astype(jnp.int32))
    y1 = jnp.minimum(y0 + 1, hm1.astype(jnp.int32))
    wvec = wvec_ref[0][None, :]
    base = b * (NHEADS * I_TOTAL) + comb_ref[0][None, :]
    i00 = base + y0 * wvec + x0
    i01 = base + y0 * wvec + x1
    i10 = base + y1 * wvec + x0
    i11 = base + y1 * wvec + x1
    gx = 1.0 - fx
    gy = 1.0 - fy
    w00 = aw * gy * gx
    w01 = aw * gy * fx
    w10 = aw * fy * gx
    w11 = aw * fy * fx
    for t, (iv, wv) in enumerate(((i00, w00), (i01, w01), (i10, w10), (i11, w11))):
        for k in range(4):
            idx_ref[0, :, t * 4 + k, :] = iv[:, k * 128:(k + 1) * 128]
        wts_ref[0, :, pl.ds(t * LANES, LANES)] = wv


def _points(queries, reference_points, Wq, bq):
    QB = 128
    Wq3 = Wq.reshape(EMB, LANES, 3)
    Wx = Wq3[..., 0]
    Wy = Wq3[..., 1]
    Wl = Wq3[..., 2]
    bq3 = bq.reshape(LANES, 3)
    bx = bq3[:, 0].reshape(1, LANES)
    by = bq3[:, 1].reshape(1, LANES)
    bl = bq3[:, 2].reshape(1, LANES)
    rpx = reference_points[..., 0].reshape(B * (NQ // QB), 1, QB)
    rpy = reference_points[..., 1].reshape(B * (NQ // QB), 1, QB)
    grid = (B, NQ // QB)
    full = lambda b, i: (0, 0)
    idx, wts = pl.pallas_call(
        _points_body,
        grid=grid,
        in_specs=[
            pl.BlockSpec((1, QB, EMB), lambda b, i: (b, i, 0)),
            pl.BlockSpec((1, 1, QB), lambda b, i: (b * (NQ // QB) + i, 0, 0)),
            pl.BlockSpec((1, 1, QB), lambda b, i: (b * (NQ // QB) + i, 0, 0)),
            pl.BlockSpec((EMB, LANES), full),
            pl.BlockSpec((EMB, LANES), full),
            pl.BlockSpec((EMB, LANES), full),
            pl.BlockSpec((1, LANES), full),
            pl.BlockSpec((1, LANES), full),
            pl.BlockSpec((1, LANES), full),
            pl.BlockSpec((1, LANES), full),
            pl.BlockSpec((1, LANES), full),
            pl.BlockSpec((1, LANES), full),
            pl.BlockSpec((1, LANES), full),
        ],
        out_specs=[
            pl.BlockSpec((1, QB, 16, 128), lambda b, i: (b, i, 0, 0)),
            pl.BlockSpec((1, QB, NROWS), lambda b, i: (b, i, 0)),
        ],
        out_shape=[
            jax.ShapeDtypeStruct((B, NQ, 16, 128), jnp.int32),
            jax.ShapeDtypeStruct((B, NQ, NROWS), jnp.float32),
        ],
    )(queries, rpx, rpy, Wx, Wy, Wl, bx, by, bl,
      jnp.asarray(_WM1).reshape(1, LANES), jnp.asarray(_HM1).reshape(1, LANES),
      jnp.asarray(_WVEC).reshape(1, LANES), jnp.asarray(_COMB).reshape(1, LANES))
    return idx.reshape(NPTS, 16, 128), wts.reshape(NPTS, NROWS)


def _sc_body(table_hbm, idx_hbm, wts_hbm, out_hbm, idxv, wtsv, rowsv, outv,
             sem_rows0, sem_rows1, sem_idx, sem_wts0, sem_wts1,
             sem_out0, sem_out1):
    wid = lax.axis_index("s") * 2 + lax.axis_index("c")
    base = wid * PTS_PER_W
    sem_rows = (sem_rows0, sem_rows1)
    sem_wts = (sem_wts0, sem_wts1)
    sem_out = (sem_out0, sem_out1)

    def clamp(pt):
        return jnp.minimum(pt, NPTS - 1)

    def idx_copy(pt, s):
        return pltpu.make_async_copy(idx_hbm.at[clamp(pt)], idxv.at[s], sem_idx)

    def wts_copy(pt, s):
        return pltpu.make_async_copy(wts_hbm.at[clamp(pt)], wtsv.at[s], sem_wts[s])

    def gathers(s):
        return [
            pltpu.make_async_copy(
                table_hbm.at[idxv.at[s, j]],
                rowsv.at[s, pl.ds(j * 128, 128), :],
                sem_rows[s],
            )
            for j in range(16)
        ]

    def start(cs):
        for cp in cs:
            cp.start()

    def wait(cs):
        for cp in cs:
            cp.wait()

    def out_copy(pt, s):
        return pltpu.make_async_copy(outv.at[s], out_hbm.at[clamp(pt)], sem_out[s])

    def compute(pt, s, k):
        @pl.when(k > 0)
        def _():
            out_copy(pt - 2, s).wait()

        def h_body(h, _):
            hb = h * (NLEVELS * NPOINTS)
            acc = [jnp.zeros((16,), jnp.float32) for _ in range(4)]
            for t in range(4):
                for g in range(2):
                    jb = t * LANES + hb + g * 16
                    wv = wtsv[s, pl.ds(jb, 16)]
                    for e in range(16):
                        w = wv[e]
                        j = jb + e
                        rev, rod = plsc.unpack(
                            rowsv[s, j, :], format=plsc.PackFormat.INTERLEAVED)
                        kk = (e & 1) * 2
                        acc[kk] = acc[kk] + w * rev
                        acc[kk + 1] = acc[kk + 1] + w * rod
            outv[s, pl.ds(h * C, 16)] = acc[0] + acc[2]
            outv[s, pl.ds(h * C + 16, 16)] = acc[1] + acc[3]
            return 0

        lax.fori_loop(0, NHEADS, h_body, 0)
        out_copy(pt, s).start()

    idx_copy(base, 0).start()
    wts_copy(base, 0).start()
    wts_copy(base + 1, 1).start()
    idx_copy(base, 0).wait()
    start(gathers(0))
    idx_copy(base + 1, 1).start()

    def pair_body(k, _):
        a = base + 2 * k
        idx_copy(a + 1, 1).wait()
        start(gathers(1))
        wait(gathers(0))
        idx_copy(a + 2, 0).start()
        wts_copy(a, 0).wait()
        compute(a, 0, k)
        wts_copy(a + 2, 0).start()
        idx_copy(a + 2, 0).wait()
        start(gathers(0))
        wait(gathers(1))
        idx_copy(a + 3, 1).start()
        wts_copy(a + 1, 1).wait()
        compute(a + 1, 1, k)
        wts_copy(a + 3, 1).start()
        return 0

    lax.fori_loop(0, PTS_PER_W // 2, pair_body, 0)
    last = base + PTS_PER_W - 1
    wait(gathers(0))
    idx_copy(last + 2, 1).wait()
    wts_copy(last + 1, 0).wait()
    wts_copy(last + 2, 1).wait()
    out_copy(last - 1, 0).wait()
    out_copy(last, 1).wait()


def _sc_gather(table, idx, wts):
    mesh = plsc.VectorSubcoreMesh(core_axis_name="c", subcore_axis_name="s")
    f = functools.partial(
        pl.kernel,
        mesh=mesh,
        compiler_params=pltpu.CompilerParams(use_tc_tiling_on_sc=False,
                                             needs_layout_passes=False),
        out_type=jax.ShapeDtypeStruct((NPTS, HID), jnp.float32),
        scratch_types=[
            pltpu.VMEM((2, 16, 128), jnp.int32),
            pltpu.VMEM((2, NROWS), jnp.float32),
            pltpu.VMEM((2, NROWS, C), jnp.bfloat16),
            pltpu.VMEM((2, HID), jnp.float32),
            pltpu.SemaphoreType.DMA,
            pltpu.SemaphoreType.DMA,
            pltpu.SemaphoreType.DMA,
            pltpu.SemaphoreType.DMA,
            pltpu.SemaphoreType.DMA,
            pltpu.SemaphoreType.DMA,
            pltpu.SemaphoreType.DMA,
        ],
    )(_sc_body)
    return f(table.reshape(B * NHEADS * I_TOTAL, C), idx, wts)


def _proj_body(x_ref, wo_ref, bo_ref, out_ref):
    out_ref[...] = (jnp.dot(x_ref[...], wo_ref[...],
                            preferred_element_type=jnp.float32)
                    + bo_ref[0][None, :])


_kk = np.tile(np.arange(C), NHEADS)
_hh = np.repeat(np.arange(NHEADS), C) * C
_PERM = (_hh + np.where(_kk < 16, 2 * _kk, 2 * (_kk - 16) + 1)).astype(np.int32)


def _out_proj(acc, Wo, bo):
    MB = 256
    return pl.pallas_call(
        _proj_body,
        grid=(NPTS // MB,),
        in_specs=[
            pl.BlockSpec((MB, HID), lambda i: (i, 0)),
            pl.BlockSpec((HID, EMB), lambda i: (0, 0)),
            pl.BlockSpec((1, EMB), lambda i: (0, 0)),
        ],
        out_specs=pl.BlockSpec((MB, EMB), lambda i: (i, 0)),
        out_shape=jax.ShapeDtypeStruct((NPTS, EMB), jnp.float32),
    )(acc, Wo[jnp.asarray(_PERM)], bo.reshape(1, EMB))


def kernel(img, shapes, queries, reference_points, Wi, bi, Wq, bq, Wo, bo):
    table = _imgp_table(img, Wi, bi)
    idx, wts = _points(queries, reference_points, Wq, bq)
    acc = _sc_gather(table, idx, wts)
    out = _out_proj(acc, Wo, bo)
    return out.reshape(B, NQ, EMB)


# speedup vs baseline: 4752.0668x; 1.0007x over previous
import functools

import jax
import jax.numpy as jnp
import numpy as np
from jax import lax
from jax.experimental import pallas as pl
from jax.experimental.pallas import tpu as pltpu
from jax.experimental.pallas import tpu_sc as plsc

EMB = 512
HID = 512
NHEADS = 16
NLEVELS = 4
NPOINTS = 8
B = 2
NQ = 1024
LEVEL_SHAPES = [[64, 64], [32, 32], [16, 16], [8, 8]]
I_TOTAL = sum(h * w for h, w in LEVEL_SHAPES)
C = HID // NHEADS
LANES = NHEADS * NLEVELS * NPOINTS
NPTS = B * NQ
NROWS = 4 * LANES
NWORK = 32
PTS_PER_W = NPTS // NWORK

_shapes_np = np.array(LEVEL_SHAPES, np.int32)
_sizes = _shapes_np[:, 0] * _shapes_np[:, 1]
_lev_start = np.concatenate([[0], np.cumsum(_sizes)[:-1]]).astype(np.int32)
_lane_l = (np.arange(LANES) // NPOINTS) % NLEVELS
_lane_h = np.arange(LANES) // (NLEVELS * NPOINTS)
_WM1 = (_shapes_np[_lane_l, 1] - 1).astype(np.float32)
_HM1 = (_shapes_np[_lane_l, 0] - 1).astype(np.float32)
_WVEC = _shapes_np[_lane_l, 1].astype(np.int32)
_COMB = (_lane_h * I_TOTAL + _lev_start[_lane_l]).astype(np.int32)


def _imgp_body(img_ref, wi_ref, bi_ref, out_ref):
    x = img_ref[0]
    y = jnp.dot(x, wi_ref[...], preferred_element_type=jnp.float32)
    y = y + bi_ref[0][None, :]
    yb = y.astype(jnp.bfloat16)
    for h in range(NHEADS):
        out_ref[0, h, :, :] = yb[:, h * C:(h + 1) * C]


def _imgp_table(img, Wi, bi):
    IB = 544
    grid = (B, I_TOTAL // IB)
    return pl.pallas_call(
        _imgp_body,
        grid=grid,
        in_specs=[
            pl.BlockSpec((1, IB, EMB), lambda b, i: (b, i, 0)),
            pl.BlockSpec((EMB, HID), lambda b, i: (0, 0)),
            pl.BlockSpec((1, HID), lambda b, i: (0, 0)),
        ],
        out_specs=pl.BlockSpec((1, NHEADS, IB, C), lambda b, i: (b, 0, i, 0)),
        out_shape=jax.ShapeDtypeStruct((B, NHEADS, I_TOTAL, C), jnp.bfloat16),
    )(img, Wi, bi.reshape(1, HID))


def _points_body(q_ref, rpx_ref, rpy_ref, wx_ref, wy_ref, wl_ref, bx_ref,
                 by_ref, bl_ref, wm1_ref, hm1_ref, wvec_ref, comb_ref,
                 idx_ref, wts_ref):
    b = pl.program_id(0)
    q = q_ref[0]
    rpx_row = rpx_ref[0, 0]
    rpy_row = rpy_ref[0, 0]
    ox = jnp.dot(q, wx_ref[...], preferred_element_type=jnp.float32) + bx_ref[0][None, :]
    oy = jnp.dot(q, wy_ref[...], preferred_element_type=jnp.float32) + by_ref[0][None, :]
    lg = jnp.dot(q, wl_ref[...], preferred_element_type=jnp.float32) + bl_ref[0][None, :]
    QB = ox.shape[0]
    lg3 = lg.reshape(QB, NHEADS, NLEVELS * NPOINTS)
    m = jnp.max(lg3, axis=2, keepdims=True)
    e = jnp.exp(lg3 - m)
    aw = (e / jnp.sum(e, axis=2, keepdims=True)).reshape(QB, LANES)

    wm1 = wm1_ref[0][None, :]
    hm1 = hm1_ref[0][None, :]
    spx = rpx_row[:, None] + ox
    spy = rpy_row[:, None] + oy
    x = jnp.clip(spx * wm1, 0.0, wm1)
    y = jnp.clip(spy * hm1, 0.0, hm1)
    x0f = jnp.floor(x)
    y0f = jnp.floor(y)
    fx = x - x0f
    fy = y - y0f
    x0 = x0f.astype(jnp.int32)
    y0 = y0f.astype(jnp.int32)
    x1 = jnp.minimum(x0 + 1, wm1.astype(jnp.int32))
    y1 = jnp.minimum(y0 + 1, hm1.astype(jnp.int32))
    wvec = wvec_ref[0][None, :]
    base = b * (NHEADS * I_TOTAL) + comb_ref[0][None, :]
    i00 = base + y0 * wvec + x0
    i01 = base + y0 * wvec + x1
    i10 = base + y1 * wvec + x0
    i11 = base + y1 * wvec + x1
    gx = 1.0 - fx
    gy = 1.0 - fy
    w00 = aw * gy * gx
    w01 = aw * gy * fx
    w10 = aw * fy * gx
    w11 = aw * fy * fx
    for t, (iv, wv) in enumerate(((i00, w00), (i01, w01), (i10, w10), (i11, w11))):
        for k in range(4):
            idx_ref[0, :, t * 4 + k, :] = iv[:, k * 128:(k + 1) * 128]
        wts_ref[0, :, pl.ds(t * LANES, LANES)] = wv


def _points(queries, reference_points, Wq, bq):
    QB = 128
    Wq3 = Wq.reshape(EMB, LANES, 3)
    Wx = Wq3[..., 0]
    Wy = Wq3[..., 1]
    Wl = Wq3[..., 2]
    bq3 = bq.reshape(LANES, 3)
    bx = bq3[:, 0].reshape(1, LANES)
    by = bq3[:, 1].reshape(1, LANES)
    bl = bq3[:, 2].reshape(1, LANES)
    rpx = reference_points[..., 0].reshape(B * (NQ // QB), 1, QB)
    rpy = reference_points[..., 1].reshape(B * (NQ // QB), 1, QB)
    grid = (B, NQ // QB)
    full = lambda b, i: (0, 0)
    idx, wts = pl.pallas_call(
        _points_body,
        grid=grid,
        in_specs=[
            pl.BlockSpec((1, QB, EMB), lambda b, i: (b, i, 0)),
            pl.BlockSpec((1, 1, QB), lambda b, i: (b * (NQ // QB) + i, 0, 0)),
            pl.BlockSpec((1, 1, QB), lambda b, i: (b * (NQ // QB) + i, 0, 0)),
            pl.BlockSpec((EMB, LANES), full),
            pl.BlockSpec((EMB, LANES), full),
            pl.BlockSpec((EMB, LANES), full),
            pl.BlockSpec((1, LANES), full),
            pl.BlockSpec((1, LANES), full),
            pl.BlockSpec((1, LANES), full),
            pl.BlockSpec((1, LANES), full),
            pl.BlockSpec((1, LANES), full),
            pl.BlockSpec((1, LANES), full),
            pl.BlockSpec((1, LANES), full),
        ],
        out_specs=[
            pl.BlockSpec((1, QB, 16, 128), lambda b, i: (b, i, 0, 0)),
            pl.BlockSpec((1, QB, NROWS), lambda b, i: (b, i, 0)),
        ],
        out_shape=[
            jax.ShapeDtypeStruct((B, NQ, 16, 128), jnp.int32),
            jax.ShapeDtypeStruct((B, NQ, NROWS), jnp.float32),
        ],
    )(queries, rpx, rpy, Wx, Wy, Wl, bx, by, bl,
      jnp.asarray(_WM1).reshape(1, LANES), jnp.asarray(_HM1).reshape(1, LANES),
      jnp.asarray(_WVEC).reshape(1, LANES), jnp.asarray(_COMB).reshape(1, LANES))
    return idx.reshape(NPTS, 16, 128), wts.reshape(NPTS, NROWS)


def _sc_body(table_hbm, idx_hbm, wts_hbm, out_hbm, idxv, wtsv, rowsv, outv,
             sem_rows0, sem_rows1, sem_idx, sem_wts0, sem_wts1,
             sem_out0, sem_out1):
    wid = lax.axis_index("s") * 2 + lax.axis_index("c")
    base = wid * PTS_PER_W
    sem_rows = (sem_rows0, sem_rows1)
    sem_wts = (sem_wts0, sem_wts1)
    sem_out = (sem_out0, sem_out1)

    def clamp(pt):
        return jnp.minimum(pt, NPTS - 1)

    def idx_copy(pt, s):
        return pltpu.make_async_copy(idx_hbm.at[clamp(pt)], idxv.at[s], sem_idx)

    def wts_copy(pt, s):
        return pltpu.make_async_copy(wts_hbm.at[clamp(pt)], wtsv.at[s], sem_wts[s])

    def gathers(s):
        return [
            pltpu.make_async_copy(
                table_hbm.at[idxv.at[s, j]],
                rowsv.at[s, pl.ds(j * 128, 128), :],
                sem_rows[s],
            )
            for j in range(16)
        ]

    def start(cs):
        for cp in cs:
            cp.start()

    def wait(cs):
        for cp in cs:
            cp.wait()

    def out_copy(pt, s):
        return pltpu.make_async_copy(outv.at[s], out_hbm.at[clamp(pt)], sem_out[s])

    def compute(pt, s, k):
        @pl.when(k > 0)
        def _():
            out_copy(pt - 2, s).wait()

        def h_body(h, _):
            hb = h * (NLEVELS * NPOINTS)
            acc = [jnp.zeros((16,), jnp.float32) for _ in range(4)]
            for t in range(4):
                for g in range(2):
                    jb = t * LANES + hb + g * 16
                    wv = wtsv[s, pl.ds(jb, 16)]
                    for q in range(4):
                        p = []
                        for e in range(4):
                            w = wv[q * 4 + e]
                            ws = jnp.full((16,), w, jnp.float32)
                            wb = plsc.pack(ws, ws,
                                           format=plsc.PackFormat.INTERLEAVED)
                            p.append(rowsv[s, jb + q * 4 + e, :] * wb)
                        tree = (p[0] + p[1]) + (p[2] + p[3])
                        rev, rod = plsc.unpack(
                            tree, format=plsc.PackFormat.INTERLEAVED)
                        kk = (q & 1) * 2
                        acc[kk] = acc[kk] + rev
                        acc[kk + 1] = acc[kk + 1] + rod
            outv[s, pl.ds(h * C, 16)] = acc[0] + acc[2]
            outv[s, pl.ds(h * C + 16, 16)] = acc[1] + acc[3]
            return 0

        lax.fori_loop(0, NHEADS, h_body, 0)
        out_copy(pt, s).start()

    idx_copy(base, 0).start()
    wts_copy(base, 0).start()
    wts_copy(base + 1, 1).start()
    idx_copy(base, 0).wait()
    start(gathers(0))
    idx_copy(base + 1, 1).start()

    def pair_body(k, _):
        a = base + 2 * k
        idx_copy(a + 1, 1).wait()
        start(gathers(1))
        wait(gathers(0))
        idx_copy(a + 2, 0).start()
        wts_copy(a, 0).wait()
        compute(a, 0, k)
        wts_copy(a + 2, 0).start()
        idx_copy(a + 2, 0).wait()
        start(gathers(0))
        wait(gathers(1))
        idx_copy(a + 3, 1).start()
        wts_copy(a + 1, 1).wait()
        compute(a + 1, 1, k)
        wts_copy(a + 3, 1).start()
        return 0

    lax.fori_loop(0, PTS_PER_W // 2, pair_body, 0)
    last = base + PTS_PER_W - 1
    wait(gathers(0))
    idx_copy(last + 2, 1).wait()
    wts_copy(last + 1, 0).wait()
    wts_copy(last + 2, 1).wait()
    out_copy(last - 1, 0).wait()
    out_copy(last, 1).wait()


def _sc_gather(table, idx, wts):
    mesh = plsc.VectorSubcoreMesh(core_axis_name="c", subcore_axis_name="s")
    f = functools.partial(
        pl.kernel,
        mesh=mesh,
        compiler_params=pltpu.CompilerParams(use_tc_tiling_on_sc=False,
                                             needs_layout_passes=False),
        out_type=jax.ShapeDtypeStruct((NPTS, HID), jnp.float32),
        scratch_types=[
            pltpu.VMEM((2, 16, 128), jnp.int32),
            pltpu.VMEM((2, NROWS), jnp.float32),
            pltpu.VMEM((2, NROWS, C), jnp.bfloat16),
            pltpu.VMEM((2, HID), jnp.float32),
            pltpu.SemaphoreType.DMA,
            pltpu.SemaphoreType.DMA,
            pltpu.SemaphoreType.DMA,
            pltpu.SemaphoreType.DMA,
            pltpu.SemaphoreType.DMA,
            pltpu.SemaphoreType.DMA,
            pltpu.SemaphoreType.DMA,
        ],
    )(_sc_body)
    return f(table.reshape(B * NHEADS * I_TOTAL, C), idx, wts)


def _proj_body(x_ref, wo_ref, bo_ref, out_ref):
    out_ref[...] = (jnp.dot(x_ref[...], wo_ref[...],
                            preferred_element_type=jnp.float32)
                    + bo_ref[0][None, :])


_kk = np.tile(np.arange(C), NHEADS)
_hh = np.repeat(np.arange(NHEADS), C) * C
_PERM = (_hh + np.where(_kk < 16, 2 * _kk, 2 * (_kk - 16) + 1)).astype(np.int32)


def _out_proj(acc, Wo, bo):
    MB = 256
    return pl.pallas_call(
        _proj_body,
        grid=(NPTS // MB,),
        in_specs=[
            pl.BlockSpec((MB, HID), lambda i: (i, 0)),
            pl.BlockSpec((HID, EMB), lambda i: (0, 0)),
            pl.BlockSpec((1, EMB), lambda i: (0, 0)),
        ],
        out_specs=pl.BlockSpec((MB, EMB), lambda i: (i, 0)),
        out_shape=jax.ShapeDtypeStruct((NPTS, EMB), jnp.float32),
    )(acc, Wo[jnp.asarray(_PERM)], bo.reshape(1, EMB))


def kernel(img, shapes, queries, reference_points, Wi, bi, Wq, bq, Wo, bo):
    table = _imgp_table(img, Wi, bi)
    idx, wts = _points(queries, reference_points, Wq, bq)
    acc = _sc_gather(table, idx, wts)
    out = _out_proj(acc, Wo, bo)
    return out.reshape(B, NQ, EMB)


# speedup vs baseline: 5196.4048x; 1.0935x over previous
import functools

import jax
import jax.numpy as jnp
import numpy as np
from jax import lax
from jax.experimental import pallas as pl
from jax.experimental.pallas import tpu as pltpu
from jax.experimental.pallas import tpu_sc as plsc

EMB = 512
HID = 512
NHEADS = 16
NLEVELS = 4
NPOINTS = 8
B = 2
NQ = 1024
LEVEL_SHAPES = [[64, 64], [32, 32], [16, 16], [8, 8]]
I_TOTAL = sum(h * w for h, w in LEVEL_SHAPES)
C = HID // NHEADS
LANES = NHEADS * NLEVELS * NPOINTS
NPTS = B * NQ
NROWS = 4 * LANES
NWORK = 32
PTS_PER_W = NPTS // NWORK

_shapes_np = np.array(LEVEL_SHAPES, np.int32)
_sizes = _shapes_np[:, 0] * _shapes_np[:, 1]
_lev_start = np.concatenate([[0], np.cumsum(_sizes)[:-1]]).astype(np.int32)
_lane_l = (np.arange(LANES) // NPOINTS) % NLEVELS
_lane_h = np.arange(LANES) // (NLEVELS * NPOINTS)
_WM1 = (_shapes_np[_lane_l, 1] - 1).astype(np.float32)
_HM1 = (_shapes_np[_lane_l, 0] - 1).astype(np.float32)
_WVEC = _shapes_np[_lane_l, 1].astype(np.int32)
_COMB = (_lane_h * I_TOTAL + _lev_start[_lane_l]).astype(np.int32)


def _imgp_body(img_ref, imgn_ref, wi_ref, bi_ref, out_ref):
    x = img_ref[0]
    xn = imgn_ref[0]
    y = jnp.dot(x, wi_ref[...], preferred_element_type=jnp.float32)
    y = y + bi_ref[0][None, :]
    yb = y.astype(jnp.bfloat16)
    yn = jnp.dot(xn, wi_ref[...], preferred_element_type=jnp.float32)
    yn = yn + bi_ref[0][None, :]
    ybn = yn.astype(jnp.bfloat16)
    ysh = jnp.concatenate([yb[1:], ybn[:1]], axis=0)
    for h in range(NHEADS):
        out_ref[0, h, :, 0:C] = yb[:, h * C:(h + 1) * C]
        out_ref[0, h, :, C:2 * C] = ysh[:, h * C:(h + 1) * C]


def _imgp_table(img, Wi, bi):
    IB = 544
    grid = (B, I_TOTAL // IB)
    nblk = I_TOTAL // 8 - 1
    return pl.pallas_call(
        _imgp_body,
        grid=grid,
        in_specs=[
            pl.BlockSpec((1, IB, EMB), lambda b, i: (b, i, 0)),
            pl.BlockSpec((1, 8, EMB),
                         lambda b, i: (b, jnp.minimum((i + 1) * (IB // 8), nblk), 0)),
            pl.BlockSpec((EMB, HID), lambda b, i: (0, 0)),
            pl.BlockSpec((1, HID), lambda b, i: (0, 0)),
        ],
        out_specs=pl.BlockSpec((1, NHEADS, IB, 2 * C), lambda b, i: (b, 0, i, 0)),
        out_shape=jax.ShapeDtypeStruct((B, NHEADS, I_TOTAL, 2 * C), jnp.bfloat16),
    )(img, img, Wi, bi.reshape(1, HID))


def _points_body(q_ref, rpx_ref, rpy_ref, wx_ref, wy_ref, wl_ref, bx_ref,
                 by_ref, bl_ref, wm1_ref, hm1_ref, wvec_ref, comb_ref,
                 idx_ref, wts_ref):
    b = pl.program_id(0)
    q = q_ref[0]
    rpx_row = rpx_ref[0, 0]
    rpy_row = rpy_ref[0, 0]
    ox = jnp.dot(q, wx_ref[...], preferred_element_type=jnp.float32) + bx_ref[0][None, :]
    oy = jnp.dot(q, wy_ref[...], preferred_element_type=jnp.float32) + by_ref[0][None, :]
    lg = jnp.dot(q, wl_ref[...], preferred_element_type=jnp.float32) + bl_ref[0][None, :]
    QB = ox.shape[0]
    lg3 = lg.reshape(QB, NHEADS, NLEVELS * NPOINTS)
    m = jnp.max(lg3, axis=2, keepdims=True)
    e = jnp.exp(lg3 - m)
    aw = (e / jnp.sum(e, axis=2, keepdims=True)).reshape(QB, LANES)

    wm1 = wm1_ref[0][None, :]
    hm1 = hm1_ref[0][None, :]
    spx = rpx_row[:, None] + ox
    spy = rpy_row[:, None] + oy
    x = jnp.clip(spx * wm1, 0.0, wm1)
    y = jnp.clip(spy * hm1, 0.0, hm1)
    x0f = jnp.minimum(jnp.floor(x), wm1 - 1.0)
    y0f = jnp.minimum(jnp.floor(y), hm1 - 1.0)
    fx = x - x0f
    fy = y - y0f
    x0 = x0f.astype(jnp.int32)
    y0 = y0f.astype(jnp.int32)
    wvec = wvec_ref[0][None, :]
    base = b * (NHEADS * I_TOTAL) + comb_ref[0][None, :]
    i0 = base + y0 * wvec + x0
    i1 = i0 + wvec
    gx = 1.0 - fx
    gy = 1.0 - fy
    w00 = aw * gy * gx
    w01 = aw * gy * fx
    w10 = aw * fy * gx
    w11 = aw * fy * fx
    for t, iv in enumerate((i0, i1)):
        for k in range(4):
            idx_ref[0, :, t * 4 + k, :] = iv[:, k * 128:(k + 1) * 128]
    for t, wv in enumerate((w00, w01, w10, w11)):
        wts_ref[0, :, pl.ds(t * LANES, LANES)] = wv


def _points(queries, reference_points, Wq, bq):
    QB = 128
    Wq3 = Wq.reshape(EMB, LANES, 3)
    Wx = Wq3[..., 0]
    Wy = Wq3[..., 1]
    Wl = Wq3[..., 2]
    bq3 = bq.reshape(LANES, 3)
    bx = bq3[:, 0].reshape(1, LANES)
    by = bq3[:, 1].reshape(1, LANES)
    bl = bq3[:, 2].reshape(1, LANES)
    rpx = reference_points[..., 0].reshape(B * (NQ // QB), 1, QB)
    rpy = reference_points[..., 1].reshape(B * (NQ // QB), 1, QB)
    grid = (B, NQ // QB)
    full = lambda b, i: (0, 0)
    idx, wts = pl.pallas_call(
        _points_body,
        grid=grid,
        in_specs=[
            pl.BlockSpec((1, QB, EMB), lambda b, i: (b, i, 0)),
            pl.BlockSpec((1, 1, QB), lambda b, i: (b * (NQ // QB) + i, 0, 0)),
            pl.BlockSpec((1, 1, QB), lambda b, i: (b * (NQ // QB) + i, 0, 0)),
            pl.BlockSpec((EMB, LANES), full),
            pl.BlockSpec((EMB, LANES), full),
            pl.BlockSpec((EMB, LANES), full),
            pl.BlockSpec((1, LANES), full),
            pl.BlockSpec((1, LANES), full),
            pl.BlockSpec((1, LANES), full),
            pl.BlockSpec((1, LANES), full),
            pl.BlockSpec((1, LANES), full),
            pl.BlockSpec((1, LANES), full),
            pl.BlockSpec((1, LANES), full),
        ],
        out_specs=[
            pl.BlockSpec((1, QB, 8, 128), lambda b, i: (b, i, 0, 0)),
            pl.BlockSpec((1, QB, NROWS), lambda b, i: (b, i, 0)),
        ],
        out_shape=[
            jax.ShapeDtypeStruct((B, NQ, 8, 128), jnp.int32),
            jax.ShapeDtypeStruct((B, NQ, NROWS), jnp.float32),
        ],
    )(queries, rpx, rpy, Wx, Wy, Wl, bx, by, bl,
      jnp.asarray(_WM1).reshape(1, LANES), jnp.asarray(_HM1).reshape(1, LANES),
      jnp.asarray(_WVEC).reshape(1, LANES), jnp.asarray(_COMB).reshape(1, LANES))
    return idx.reshape(NPTS, 8, 128), wts.reshape(NPTS, NROWS)


def _sc_body(table_hbm, idx_hbm, wts_hbm, out_hbm, idxv, wtsv, rowsv, outv,
             sem_rows0, sem_rows1, sem_idx, sem_wts0, sem_wts1,
             sem_out0, sem_out1):
    wid = lax.axis_index("s") * 2 + lax.axis_index("c")
    base = wid * PTS_PER_W
    sem_rows = (sem_rows0, sem_rows1)
    sem_wts = (sem_wts0, sem_wts1)
    sem_out = (sem_out0, sem_out1)

    def clamp(pt):
        return jnp.minimum(pt, NPTS - 1)

    def idx_copy(pt, s):
        return pltpu.make_async_copy(idx_hbm.at[clamp(pt)], idxv.at[s], sem_idx)

    def wts_copy(pt, s):
        return pltpu.make_async_copy(wts_hbm.at[clamp(pt)], wtsv.at[s], sem_wts[s])

    def gathers(s):
        return [
            pltpu.make_async_copy(
                table_hbm.at[idxv.at[s, j]],
                rowsv.at[s, pl.ds(j * 128, 128), :],
                sem_rows[s],
            )
            for j in range(8)
        ]

    def start(cs):
        for cp in cs:
            cp.start()

    def wait(cs):
        for cp in cs:
            cp.wait()

    def out_copy(pt, s):
        return pltpu.make_async_copy(outv.at[s], out_hbm.at[clamp(pt)], sem_out[s])

    def compute(pt, s, k):
        @pl.when(k > 0)
        def _():
            out_copy(pt - 2, s).wait()

        def h_body(h, _):
            hb = h * (NLEVELS * NPOINTS)
            acc = [jnp.zeros((16,), jnp.float32) for _ in range(4)]
            for t in range(2):
                for g in range(2):
                    sb = t * LANES + hb + g * 16
                    wlv = wtsv[s, pl.ds(2 * t * LANES + hb + g * 16, 16)]
                    wrv = wtsv[s, pl.ds((2 * t + 1) * LANES + hb + g * 16, 16)]
                    for q in range(4):
                        p = []
                        for e in range(4):
                            j = q * 4 + e
                            wl = wlv[j]
                            wr = wrv[j]
                            wsl = jnp.full((16,), wl, jnp.float32)
                            wbl = plsc.pack(wsl, wsl,
                                            format=plsc.PackFormat.INTERLEAVED)
                            wsr = jnp.full((16,), wr, jnp.float32)
                            wbr = plsc.pack(wsr, wsr,
                                            format=plsc.PackFormat.INTERLEAVED)
                            p.append(rowsv[s, sb + j, pl.ds(0, 32)] * wbl
                                     + rowsv[s, sb + j, pl.ds(32, 32)] * wbr)
                        tree = (p[0] + p[1]) + (p[2] + p[3])
                        rev, rod = plsc.unpack(
                            tree, format=plsc.PackFormat.INTERLEAVED)
                        kk = (q & 1) * 2
                        acc[kk] = acc[kk] + rev
                        acc[kk + 1] = acc[kk + 1] + rod
            outv[s, pl.ds(h * C, 16)] = acc[0] + acc[2]
            outv[s, pl.ds(h * C + 16, 16)] = acc[1] + acc[3]
            return 0

        lax.fori_loop(0, NHEADS, h_body, 0)
        out_copy(pt, s).start()

    idx_copy(base, 0).start()
    wts_copy(base, 0).start()
    wts_copy(base + 1, 1).start()
    idx_copy(base, 0).wait()
    start(gathers(0))
    idx_copy(base + 1, 1).start()

    def pair_body(k, _):
        a = base + 2 * k
        idx_copy(a + 1, 1).wait()
        start(gathers(1))
        wait(gathers(0))
        idx_copy(a + 2, 0).start()
        wts_copy(a, 0).wait()
        compute(a, 0, k)
        wts_copy(a + 2, 0).start()
        idx_copy(a + 2, 0).wait()
        start(gathers(0))
        wait(gathers(1))
        idx_copy(a + 3, 1).start()
        wts_copy(a + 1, 1).wait()
        compute(a + 1, 1, k)
        wts_copy(a + 3, 1).start()
        return 0

    lax.fori_loop(0, PTS_PER_W // 2, pair_body, 0)
    last = base + PTS_PER_W - 1
    wait(gathers(0))
    idx_copy(last + 2, 1).wait()
    wts_copy(last + 1, 0).wait()
    wts_copy(last + 2, 1).wait()
    out_copy(last - 1, 0).wait()
    out_copy(last, 1).wait()


def _sc_gather(table, idx, wts):
    mesh = plsc.VectorSubcoreMesh(core_axis_name="c", subcore_axis_name="s")
    f = functools.partial(
        pl.kernel,
        mesh=mesh,
        compiler_params=pltpu.CompilerParams(use_tc_tiling_on_sc=False,
                                             needs_layout_passes=False),
        out_type=jax.ShapeDtypeStruct((NPTS, HID), jnp.float32),
        scratch_types=[
            pltpu.VMEM((2, 8, 128), jnp.int32),
            pltpu.VMEM((2, NROWS), jnp.float32),
            pltpu.VMEM((2, NROWS // 2, 2 * C), jnp.bfloat16),
            pltpu.VMEM((2, HID), jnp.float32),
            pltpu.SemaphoreType.DMA,
            pltpu.SemaphoreType.DMA,
            pltpu.SemaphoreType.DMA,
            pltpu.SemaphoreType.DMA,
            pltpu.SemaphoreType.DMA,
            pltpu.SemaphoreType.DMA,
            pltpu.SemaphoreType.DMA,
        ],
    )(_sc_body)
    return f(table.reshape(B * NHEADS * I_TOTAL, 2 * C), idx, wts)


def _proj_body(x_ref, wo_ref, bo_ref, out_ref):
    out_ref[...] = (jnp.dot(x_ref[...], wo_ref[...],
                            preferred_element_type=jnp.float32)
                    + bo_ref[0][None, :])


_kk = np.tile(np.arange(C), NHEADS)
_hh = np.repeat(np.arange(NHEADS), C) * C
_PERM = (_hh + np.where(_kk < 16, 2 * _kk, 2 * (_kk - 16) + 1)).astype(np.int32)


def _out_proj(acc, Wo, bo):
    MB = 256
    return pl.pallas_call(
        _proj_body,
        grid=(NPTS // MB,),
        in_specs=[
            pl.BlockSpec((MB, HID), lambda i: (i, 0)),
            pl.BlockSpec((HID, EMB), lambda i: (0, 0)),
            pl.BlockSpec((1, EMB), lambda i: (0, 0)),
        ],
        out_specs=pl.BlockSpec((MB, EMB), lambda i: (i, 0)),
        out_shape=jax.ShapeDtypeStruct((NPTS, EMB), jnp.float32),
    )(acc, Wo[jnp.asarray(_PERM)], bo.reshape(1, EMB))


def kernel(img, shapes, queries, reference_points, Wi, bi, Wq, bq, Wo, bo):
    table = _imgp_table(img, Wi, bi)
    idx, wts = _points(queries, reference_points, Wq, bq)
    acc = _sc_gather(table, idx, wts)
    out = _out_proj(acc, Wo, bo)
    return out.reshape(B, NQ, EMB)
